# Initial kernel scaffold; baseline (speedup 1.0000x reference)
import jax
import jax.numpy as jnp
from jax.experimental import pallas as pl


def kernel(tokens, table):
    raise NotImplementedError("write your pallas kernel here")


# speedup vs baseline: 4.0398x; 4.0398x over previous
import functools
import math

import jax
import jax.numpy as jnp
from jax import lax
from jax.experimental import pallas as pl
from jax.experimental.pallas import tpu as pltpu
from jax.experimental.pallas import tpu_sc as plsc

VOCAB_EMB = 32
SCALE = math.sqrt(VOCAB_EMB)

NC, NS = 2, 16
NW = NC * NS

IDXW = 128
K = 8
CHUNK = IDXW * K


def _gather_scale(table, tok2d, n_chunks):
    B = tok2d.shape[0] * IDXW
    b_per_w = B // NW
    rows_per_w = b_per_w // IDXW

    mesh = plsc.VectorSubcoreMesh(core_axis_name="c", subcore_axis_name="s")

    @functools.partial(
        pl.kernel,
        out_type=jax.ShapeDtypeStruct((B, VOCAB_EMB), jnp.float32),
        mesh=mesh,
        compiler_params=pltpu.CompilerParams(use_tc_tiling_on_sc=False),
        scratch_types=[
            pltpu.VMEM((K, IDXW), jnp.int32),
            pltpu.VMEM((CHUNK, VOCAB_EMB), jnp.float32),
            pltpu.SemaphoreType.DMA,
        ],
    )
    def k(table_hbm, tok_hbm, out_hbm, idx_v, rows_v, sem):
        wid = lax.axis_index("s") * NC + lax.axis_index("c")

        def chunk_body(c, _):
            row0 = wid * rows_per_w + c * K
            pltpu.sync_copy(tok_hbm.at[pl.ds(row0, K)], idx_v)
            copies = [
                pltpu.async_copy(
                    table_hbm.at[idx_v.at[j]],
                    rows_v.at[pl.ds(j * IDXW, IDXW)],
                    sem,
                )
                for j in range(K)
            ]
            for cd in copies:
                cd.wait()

            def scale_row(i, _):
                lo = rows_v[i, pl.ds(0, 16)]
                hi = rows_v[i, pl.ds(16, 16)]
                rows_v[i, pl.ds(0, 16)] = lo * SCALE
                rows_v[i, pl.ds(16, 16)] = hi * SCALE
                return 0

            lax.fori_loop(0, CHUNK, scale_row, 0)
            base = wid * b_per_w + c * CHUNK
            pltpu.sync_copy(rows_v, out_hbm.at[pl.ds(base, CHUNK)])
            return 0

        lax.fori_loop(0, n_chunks, chunk_body, 0)

    return k(table, tok2d)


def kernel(tokens, table):
    n_tok = tokens.size
    assert n_tok % (NW * CHUNK) == 0, n_tok
    n_chunks = n_tok // (NW * CHUNK)
    tok2d = tokens.reshape(n_tok // IDXW, IDXW)
    out = _gather_scale(table, tok2d, n_chunks)
    return out.reshape(tokens.shape + (VOCAB_EMB,))


# speedup vs baseline: 4.9198x; 1.2178x over previous
import functools
import math

import jax
import jax.numpy as jnp
from jax import lax
from jax.experimental import pallas as pl
from jax.experimental.pallas import tpu as pltpu
from jax.experimental.pallas import tpu_sc as plsc

EMB = 32
SCALE = math.sqrt(EMB)

NC, NS = 2, 16
NW = NC * NS

IDXW = 128
K = 8
CHUNK = IDXW * K
NBUF = 3


def _gather_scale(table, tok2d, n_chunks):
    B = tok2d.shape[0] * IDXW
    b_per_w = B // NW
    rows_per_w = b_per_w // IDXW
    n_outer = (n_chunks + NBUF - 1) // NBUF

    mesh = plsc.VectorSubcoreMesh(core_axis_name="c", subcore_axis_name="s")

    @functools.partial(
        pl.kernel,
        out_type=jax.ShapeDtypeStruct((B, EMB), jnp.float32),
        mesh=mesh,
        compiler_params=pltpu.CompilerParams(use_tc_tiling_on_sc=False),
        scratch_types=[
            pltpu.VMEM((NBUF, K, IDXW), jnp.int32),
            pltpu.VMEM((NBUF, CHUNK, EMB), jnp.float32),
            pltpu.SemaphoreType.DMA((NBUF,)),
            pltpu.SemaphoreType.DMA((NBUF,)),
        ],
    )
    def k(table_hbm, tok_hbm, out_hbm, idx_v, rows_v, sem_g, sem_w):
        wid = lax.axis_index("s") * NC + lax.axis_index("c")
        idx_row0 = wid * rows_per_w
        out_row0 = wid * b_per_w

        def stage_and_fire(c, b):
            pltpu.sync_copy(tok_hbm.at[pl.ds(idx_row0 + c * K, K)], idx_v.at[b])
            for j in range(K):
                pltpu.async_copy(
                    table_hbm.at[idx_v.at[b, j]],
                    rows_v.at[b, pl.ds(j * IDXW, IDXW)],
                    sem_g.at[b],
                )

        def wait_gather(b):
            pltpu.make_async_copy(
                table_hbm.at[pl.ds(0, CHUNK)], rows_v.at[b], sem_g.at[b]
            ).wait()

        def fire_write(c, b):
            pltpu.async_copy(
                rows_v.at[b], out_hbm.at[pl.ds(out_row0 + c * CHUNK, CHUNK)],
                sem_w.at[b],
            )

        def wait_write(b):
            pltpu.make_async_copy(
                rows_v.at[b], out_hbm.at[pl.ds(0, CHUNK)], sem_w.at[b]
            ).wait()

        for c in range(min(2, n_chunks)):
            stage_and_fire(c, c % NBUF)

        def outer(o):
            for b in range(NBUF):
                c = o * NBUF + b
                b2 = (b + 2) % NBUF

                @pl.when(c < n_chunks)
                def _():
                    wait_gather(b)

                    @pl.when(c + 2 < n_chunks)
                    def _():
                        @pl.when(c + 2 >= NBUF)
                        def _():
                            wait_write(b2)

                        stage_and_fire(c + 2, b2)

                    @pl.loop(0, CHUNK, unroll=8)
                    def _(i):
                        lo = rows_v[b, i, pl.ds(0, 16)]
                        hi = rows_v[b, i, pl.ds(16, 16)]
                        rows_v[b, i, pl.ds(0, 16)] = lo * SCALE
                        rows_v[b, i, pl.ds(16, 16)] = hi * SCALE

                    fire_write(c, b)

        pl.loop(0, n_outer)(outer)

        for c in range(max(0, n_chunks - NBUF), n_chunks):
            wait_write(c % NBUF)

    return k(table, tok2d)


def kernel(tokens, table):
    n_tok = tokens.size
    assert n_tok % (NW * CHUNK) == 0, n_tok
    n_chunks = n_tok // (NW * CHUNK)
    tok2d = tokens.reshape(n_tok // IDXW, IDXW)
    out = _gather_scale(table, tok2d, n_chunks)
    return out.reshape(tokens.shape + (EMB,))


# speedup vs baseline: 8.8715x; 1.8032x over previous
import functools
import math

import jax
import jax.numpy as jnp
from jax import lax
from jax.experimental import pallas as pl
from jax.experimental.pallas import tpu as pltpu
from jax.experimental.pallas import tpu_sc as plsc

EMB = 32
PAD = 128
SCALE = math.sqrt(EMB)

NC, NS = 2, 16
NW = NC * NS

ROWS_PER_CHUNK = 4
TOK_PER_ROW = 200
CHUNK = ROWS_PER_CHUNK * TOK_PER_ROW
NBUF = 4
GATHER_SPLITS = ((0, 128), (128, 72))


def _gather_scale(table, tokens):
    n_rows, n_cols = tokens.shape
    assert n_cols == TOK_PER_ROW
    rows_per_w = n_rows // NW
    n_chunks = rows_per_w // ROWS_PER_CHUNK

    mesh = plsc.VectorSubcoreMesh(core_axis_name="c", subcore_axis_name="s")

    @functools.partial(
        pl.kernel,
        out_type=jax.ShapeDtypeStruct((n_rows, TOK_PER_ROW, PAD), jnp.float32),
        mesh=mesh,
        compiler_params=pltpu.CompilerParams(use_tc_tiling_on_sc=False),
        scratch_types=[
            pltpu.VMEM((NBUF, ROWS_PER_CHUNK, TOK_PER_ROW), jnp.int32),
            pltpu.VMEM((NBUF, CHUNK, EMB), jnp.float32),
            pltpu.SemaphoreType.DMA((NBUF,)),
            pltpu.SemaphoreType.DMA((NBUF,)),
        ],
    )
    def k(table_hbm, tok_hbm, out_hbm, idx_v, rows_v, sem_g, sem_w):
        wid = lax.axis_index("s") * NC + lax.axis_index("c")
        tr_base = wid * rows_per_w

        def stage_and_fire(c, b):
            tr0 = tr_base + c * ROWS_PER_CHUNK
            pltpu.sync_copy(tok_hbm.at[pl.ds(tr0, ROWS_PER_CHUNK)], idx_v.at[b])
            for r in range(ROWS_PER_CHUNK):
                for off, n in GATHER_SPLITS:
                    pltpu.async_copy(
                        table_hbm.at[idx_v.at[b, r, pl.ds(off, n)]],
                        rows_v.at[b, pl.ds(r * TOK_PER_ROW + off, n)],
                        sem_g.at[b],
                    )

        def wait_gather(b):
            pltpu.make_async_copy(
                table_hbm.at[pl.ds(0, CHUNK)], rows_v.at[b], sem_g.at[b]
            ).wait()

        def fire_write(c, b):
            tr0 = tr_base + c * ROWS_PER_CHUNK
            for r in range(ROWS_PER_CHUNK):
                pltpu.async_copy(
                    rows_v.at[b, pl.ds(r * TOK_PER_ROW, TOK_PER_ROW)],
                    out_hbm.at[tr0 + r, pl.ds(0, TOK_PER_ROW), pl.ds(0, EMB)],
                    sem_w.at[b],
                )

        def wait_write(b):
            pltpu.make_async_copy(
                rows_v.at[b], table_hbm.at[pl.ds(0, CHUNK)], sem_w.at[b]
            ).wait()

        for c in range(min(2, n_chunks)):
            stage_and_fire(c, c % NBUF)

        def outer(o):
            for b in range(NBUF):
                c = o * NBUF + b
                b2 = (b + 2) % NBUF

                @pl.when(c < n_chunks)
                def _():
                    wait_gather(b)

                    @pl.when(c + 2 < n_chunks)
                    def _():
                        @pl.when(c + 2 >= NBUF)
                        def _():
                            wait_write(b2)

                        stage_and_fire(c + 2, b2)

                    @pl.loop(0, CHUNK, unroll=8)
                    def _(i):
                        lo = rows_v[b, i, pl.ds(0, 16)]
                        hi = rows_v[b, i, pl.ds(16, 16)]
                        rows_v[b, i, pl.ds(0, 16)] = lo * SCALE
                        rows_v[b, i, pl.ds(16, 16)] = hi * SCALE

                    fire_write(c, b)

        pl.loop(0, (n_chunks + NBUF - 1) // NBUF)(outer)

        for c in range(max(0, n_chunks - NBUF), n_chunks):
            wait_write(c % NBUF)

    return k(table, tokens)


def kernel(tokens, table):
    padded = _gather_scale(table, tokens)
    return padded[:, :, :EMB]


# speedup vs baseline: 14.5999x; 1.6457x over previous
import functools
import math

import jax
import jax.numpy as jnp
from jax import lax
from jax.experimental import pallas as pl
from jax.experimental.pallas import tpu as pltpu
from jax.experimental.pallas import tpu_sc as plsc

EMB = 32
SCALE = math.sqrt(EMB)

NC, NS = 2, 16
NW = NC * NS

TPB = 128
PR = 4
CHUNK = PR * TPB
NBUF = 3


def _gather_scale_t(table_flat, tokens, vocab):
    n_rows, n_pos = tokens.shape
    n_stripes_w = n_rows // (NW * TPB)
    subs_per_stripe = n_pos // PR
    n_sub = n_stripes_w * subs_per_stripe

    mesh = plsc.VectorSubcoreMesh(core_axis_name="c", subcore_axis_name="s")

    @functools.partial(
        pl.kernel,
        out_type=jax.ShapeDtypeStruct(
            (n_pos, EMB // 8, n_rows // TPB, 8, TPB), jnp.float32
        ),
        mesh=mesh,
        compiler_params=pltpu.CompilerParams(
            use_tc_tiling_on_sc=False, needs_layout_passes=False
        ),
        scratch_types=[
            pltpu.VMEM((TPB, n_pos + 1), jnp.int32),
            pltpu.VMEM((NBUF, CHUNK), jnp.int32),
            pltpu.VMEM((NBUF, CHUNK, EMB), jnp.float32),
            pltpu.VMEM((NBUF, PR, EMB // 8, 8, TPB + 1), jnp.float32),
            pltpu.SemaphoreType.DMA((NBUF,)),
            pltpu.SemaphoreType.DMA((NBUF,)),
        ],
    )
    def k(table_hbm, tok_hbm, out_hbm, stripe_v, cols_v, rows_v, trans_v,
          sem_g, sem_w):
        table2d = table_hbm
        wid = lax.axis_index("s") * NC + lax.axis_index("c")
        lanes = lax.iota(jnp.int32, 16)
        cb_lo, ci_lo = lanes >> 3, lanes & 7
        cb_hi = cb_lo + 2

        def fire(q, b):
            stripe = q // subs_per_stripe
            r0 = (q % subs_per_stripe) * PR
            t0 = (wid * n_stripes_w + stripe) * TPB

            def load_stripe():
                pltpu.sync_copy(
                    tok_hbm.at[pl.ds(t0, TPB)],
                    stripe_v.at[pl.ds(0, TPB), pl.ds(0, n_pos)],
                )

            if isinstance(q, int):
                if q % subs_per_stripe == 0:
                    load_stripe()
            else:
                pl.when(q % subs_per_stripe == 0)(load_stripe)

            for j in range(PR):
                cvec = jnp.full((16,), r0 + j, dtype=jnp.int32)

                @plsc.parallel_loop(0, TPB // 16, unroll=8)
                def _(g):
                    ids = plsc.load_gather(stripe_v, [g * 16 + lanes, cvec])
                    cols_v[b, pl.ds(j * TPB + g * 16, 16)] = ids

            for j in range(PR):
                pltpu.async_copy(
                    table2d.at[cols_v.at[b, pl.ds(j * TPB, TPB)]],
                    rows_v.at[b, pl.ds(j * TPB, TPB)],
                    sem_g.at[b],
                )

        def wait_gather(b):
            pltpu.make_async_copy(
                table2d.at[pl.ds(0, CHUNK)], rows_v.at[b], sem_g.at[b]
            ).wait()

        def _trans_src(b):
            return trans_v.at[
                b, pl.ds(0, PR), pl.ds(0, EMB // 8), pl.ds(0, 8), pl.ds(0, TPB)
            ]

        def fire_write(q, b):
            stripe = q // subs_per_stripe
            r0 = (q % subs_per_stripe) * PR
            tb = wid * n_stripes_w + stripe
            pltpu.async_copy(
                _trans_src(b),
                out_hbm.at[
                    pl.ds(r0, PR), pl.ds(0, EMB // 8), tb,
                    pl.ds(0, 8), pl.ds(0, TPB),
                ],
                sem_w.at[b],
            )

        def wait_write(b):
            pltpu.make_async_copy(
                _trans_src(b),
                out_hbm.at[
                    pl.ds(0, PR), pl.ds(0, EMB // 8), 0,
                    pl.ds(0, 8), pl.ds(0, TPB),
                ],
                sem_w.at[b],
            ).wait()

        for q in range(min(2, n_sub)):
            fire(q, q % NBUF)

        def outer(o):
            for b in range(NBUF):
                q = o * NBUF + b
                b2 = (b + 2) % NBUF

                @pl.when(q < n_sub)
                def _():
                    wait_gather(b)

                    @pl.when(q + 2 < n_sub)
                    def _():
                        @pl.when(q + 2 >= NBUF)
                        def _():
                            wait_write(b2)

                        fire(q + 2, b2)

                    for j in range(PR):
                        @plsc.parallel_loop(0, TPB, unroll=8)
                        def _(ti):
                            row = j * TPB + ti
                            tvec = jnp.full((16,), ti, dtype=jnp.int32)
                            lo = rows_v[b, row, pl.ds(0, 16)] * SCALE
                            hi = rows_v[b, row, pl.ds(16, 16)] * SCALE
                            plsc.store_scatter(
                                trans_v.at[b, j], [cb_lo, ci_lo, tvec], lo
                            )
                            plsc.store_scatter(
                                trans_v.at[b, j], [cb_hi, ci_lo, tvec], hi
                            )

                    fire_write(q, b)

        pl.loop(0, (n_sub + NBUF - 1) // NBUF)(outer)

        for q in range(max(0, n_sub - NBUF), n_sub):
            wait_write(q % NBUF)

    return k(table_flat, tokens)


def kernel(tokens, table):
    vocab = table.shape[0]
    n_rows, n_pos = tokens.shape
    out5 = _gather_scale_t(table, tokens, vocab)
    return out5.transpose(2, 4, 0, 1, 3).reshape(n_rows, n_pos, EMB)
